# baseline (speedup 1.0000x reference)
import functools

import jax
import jax.numpy as jnp
from jax import lax
from jax.experimental import pallas as pl
from jax.experimental.pallas import tpu as pltpu
from jax.experimental.pallas import tpu_sc as plsc

N_NODES = 10000
D_FEAT = 128
N_EDGES = 320000
K_NEG = 10
INV_T = 2.0

NC = 2
NS = 16
NW = NC * NS
EDGES_PER_WORKER = N_EDGES // NW
CHUNK = 16
CHUNKS_PER_WORKER = EDGES_PER_WORKER // CHUNK
ROWS_PER_EDGE = 2 + K_NEG
CHUNK_ROWS = CHUNK * ROWS_PER_EDGE
UNROLL = 2
NBUF = 3


def _tc_normalize(x_ref, o_ref):
    x = x_ref[...]
    n = jnp.sqrt(jnp.sum(x * x, axis=1, keepdims=True))
    o_ref[...] = (x / jnp.maximum(n, 1e-12)).astype(jnp.bfloat16)


def _tc_finalize(s_ref, p_ref, o_ref):
    tot = jnp.sum(jnp.log(s_ref[...])) - INV_T * jnp.sum(p_ref[...])
    o_ref[0, 0] = tot / N_EDGES


def _sc_body(emb_hbm, idx_hbm, out_s_hbm, out_p_hbm,
             idx_a, idx_b, idx_c, rows_a, rows_b, rows_c, os_v, op_v, emb_sp,
             isem_a, isem_b, isem_c, gsem_a, gsem_b, gsem_c):
    sid = lax.axis_index("s")
    wid = sid * NC + lax.axis_index("c")
    iota4 = lax.iota(jnp.int32, 16) * 8
    wbase = wid * EDGES_PER_WORKER

    rows_per_sub = N_NODES // NS
    pltpu.sync_copy(emb_hbm.at[pl.ds(sid * rows_per_sub, rows_per_sub)],
                    emb_sp.at[pl.ds(sid * rows_per_sub, rows_per_sub)])
    plsc.subcore_barrier()
    iota = lax.iota(jnp.int32, 16)
    row_base = [iota * ROWS_PER_EDGE + j for j in range(ROWS_PER_EDGE)]
    zero = jnp.zeros((16,), jnp.float32)
    idx_bufs = (idx_a, idx_b, idx_c)
    rows_bufs = (rows_a, rows_b, rows_c)
    isems = (isem_a, isem_b, isem_c)
    gsems = (gsem_a, gsem_b, gsem_c)

    def fire_idx(c, slot):
        pltpu.async_copy(
            idx_hbm.at[pl.ds(wbase * ROWS_PER_EDGE + c * CHUNK_ROWS,
                             CHUNK_ROWS)],
            idx_bufs[slot], isems[slot])

    def wait_idx(slot):
        pltpu.make_async_copy(idx_hbm.at[pl.ds(0, CHUNK_ROWS)],
                              idx_bufs[slot], isems[slot]).wait()

    def fire_gather(slot):
        pltpu.async_copy(emb_sp.at[idx_bufs[slot].at[pl.ds(0, 128)]],
                         rows_bufs[slot].at[pl.ds(0, 128)], gsems[slot])
        pltpu.async_copy(emb_sp.at[idx_bufs[slot].at[pl.ds(128, 64)]],
                         rows_bufs[slot].at[pl.ds(128, 64)], gsems[slot])

    def wait_gather(slot):
        pltpu.make_async_copy(emb_hbm.at[pl.ds(0, CHUNK_ROWS)],
                              rows_bufs[slot], gsems[slot]).wait()

    def compute(c, slot):
        rows = rows_bufs[slot]
        npair = D_FEAT // 2
        zero_pair = jnp.zeros((2 * 16,), jnp.bfloat16)

        def dbody(t, accs):
            out = list(accs)
            for u in range(UNROLL):
                dv = (t * UNROLL + u + iota4) & (npair - 1)
                sb = plsc.bitcast(plsc.load_gather(rows, [row_base[0], dv]),
                                  jnp.bfloat16)
                tb = plsc.bitcast(plsc.load_gather(rows, [row_base[1], dv]),
                                  jnp.bfloat16)
                out[0] = out[0] + sb * tb
                for k in range(K_NEG):
                    nb = plsc.bitcast(
                        plsc.load_gather(rows, [row_base[2 + k], dv]),
                        jnp.bfloat16)
                    out[1 + k] = out[1 + k] + sb * nb
            return tuple(out)

        accs = lax.fori_loop(0, npair // UNROLL, dbody,
                             (zero_pair,) * (1 + K_NEG))

        def pair_sum(acc):
            even, odd = plsc.unpack(acc, format=plsc.PackFormat.INTERLEAVED,
                                    preferred_element_type=jnp.float32)
            return even + odd

        accs = [pair_sum(a) for a in accs]
        pos = accs[0]
        ssum = jnp.exp(pos * INV_T)
        for k in range(K_NEG):
            ssum = ssum + jnp.exp(accs[1 + k] * INV_T)
        os_v[pl.ds(c * CHUNK, CHUNK)] = ssum
        op_v[pl.ds(c * CHUNK, CHUNK)] = pos

    fire_idx(0, 0)
    fire_idx(1, 1)
    fire_idx(2, 2)
    wait_idx(0)
    fire_gather(0)
    wait_idx(1)
    fire_gather(1)

    n_full = (CHUNKS_PER_WORKER - 1) // NBUF

    def triple_body(i, carry):
        for par in range(NBUF):
            c = i * NBUF + par
            cur = par
            nxt = (par + 2) % NBUF
            wait_gather(cur)
            if par == 0:
                fire_idx(c + NBUF, cur)
            else:
                @pl.when(i < n_full - 1)
                def _():
                    fire_idx(c + NBUF, cur)
            if par == NBUF - 1:
                @pl.when(i < n_full - 1)
                def _():
                    wait_idx(nxt)
                    fire_gather(nxt)
            else:
                wait_idx(nxt)
                fire_gather(nxt)
            compute(c, cur)
        return carry

    lax.fori_loop(0, n_full, triple_body, 0)
    wait_gather(0)
    compute(CHUNKS_PER_WORKER - 1, 0)

    pltpu.sync_copy(os_v, out_s_hbm.at[pl.ds(wbase, EDGES_PER_WORKER)])
    pltpu.sync_copy(op_v, out_p_hbm.at[pl.ds(wbase, EDGES_PER_WORKER)])


_sc_kernel = functools.partial(
    pl.kernel,
    out_type=(jax.ShapeDtypeStruct((N_EDGES,), jnp.float32),
              jax.ShapeDtypeStruct((N_EDGES,), jnp.float32)),
    mesh=plsc.VectorSubcoreMesh(core_axis_name="c", subcore_axis_name="s",
                                num_cores=NC, num_subcores=NS),
    scratch_types=(
        [pltpu.VMEM((CHUNK_ROWS,), jnp.int32)] * NBUF
        + [pltpu.VMEM((CHUNK_ROWS, D_FEAT // 2), jnp.float32)] * NBUF
        + [pltpu.VMEM((EDGES_PER_WORKER,), jnp.float32)] * 2
        + [pltpu.VMEM_SHARED((N_NODES, D_FEAT // 2), jnp.float32)]
        + [pltpu.SemaphoreType.DMA] * (2 * NBUF)
    ),
    compiler_params=pltpu.CompilerParams(needs_layout_passes=False,
                                         use_tc_tiling_on_sc=False),
)(_sc_body)


def kernel(node_embeddings, edge_index, num_neg_samples):
    del num_neg_samples
    emb_bf16 = pl.pallas_call(
        _tc_normalize,
        out_shape=jax.ShapeDtypeStruct((N_NODES, D_FEAT), jnp.bfloat16),
    )(node_embeddings)
    emb = jax.lax.bitcast_convert_type(
        emb_bf16.reshape(N_NODES, D_FEAT // 2, 2), jnp.float32)

    src = edge_index[0].astype(jnp.int32)
    dst = edge_index[1].astype(jnp.int32)
    neg = jax.random.randint(jax.random.key(42), (N_EDGES, K_NEG), 0, N_NODES)
    idx_all = jnp.concatenate(
        [src[:, None], dst[:, None], neg.astype(jnp.int32)], axis=1
    ).reshape(-1)

    s_sum, pos = _sc_kernel(emb, idx_all)

    loss2d = pl.pallas_call(
        _tc_finalize,
        out_shape=jax.ShapeDtypeStruct((1, 1), jnp.float32),
        out_specs=pl.BlockSpec(memory_space=pltpu.SMEM),
    )(s_sum.reshape(N_EDGES // 128, 128), pos.reshape(N_EDGES // 128, 128))
    return loss2d[0, 0]


# speedup vs baseline: 1.1364x; 1.1364x over previous
import functools

import jax
import jax.numpy as jnp
from jax import lax
from jax.experimental import pallas as pl
from jax.experimental.pallas import tpu as pltpu
from jax.experimental.pallas import tpu_sc as plsc

N_NODES = 10000
D_FEAT = 128
N_EDGES = 320000
K_NEG = 10
INV_T = 2.0

NC = 2
NS = 16
NW = NC * NS
EDGES_PER_WORKER = N_EDGES // NW
CHUNK = 16
CHUNKS_PER_WORKER = EDGES_PER_WORKER // CHUNK
ROWS_PER_EDGE = 2 + K_NEG
CHUNK_ROWS = CHUNK * ROWS_PER_EDGE
UNROLL = 2
NBUF = 3


def _tc_normalize(x_ref, o_ref):
    x = x_ref[...]
    n = jnp.sqrt(jnp.sum(x * x, axis=1, keepdims=True))
    o_ref[...] = (x / jnp.maximum(n, 1e-12)).astype(jnp.bfloat16)


def _tc_finalize(s_ref, p_ref, o_ref):
    tot = jnp.sum(jnp.log(s_ref[...])) - INV_T * jnp.sum(p_ref[...])
    o_ref[0, 0] = tot / N_EDGES


def _sc_body(emb_hbm, idx_hbm, out_s_hbm, out_p_hbm,
             idx_a, idx_b, idx_c, rows_a, rows_b, rows_c, os_v, op_v, emb_sp,
             isem_a, isem_b, isem_c, gsem_a, gsem_b, gsem_c):
    sid = lax.axis_index("s")
    wid = sid * NC + lax.axis_index("c")
    iota4 = lax.iota(jnp.int32, 16) * 4
    wbase = wid * EDGES_PER_WORKER

    rows_per_sub = N_NODES // NS
    pltpu.sync_copy(emb_hbm.at[pl.ds(sid * rows_per_sub, rows_per_sub)],
                    emb_sp.at[pl.ds(sid * rows_per_sub, rows_per_sub)])
    plsc.subcore_barrier()
    iota = lax.iota(jnp.int32, 16)
    row_base = [iota * ROWS_PER_EDGE + j for j in range(ROWS_PER_EDGE)]
    zero = jnp.zeros((16,), jnp.float32)
    idx_bufs = (idx_a, idx_b, idx_c)
    rows_bufs = (rows_a, rows_b, rows_c)
    isems = (isem_a, isem_b, isem_c)
    gsems = (gsem_a, gsem_b, gsem_c)

    def fire_idx(c, slot):
        pltpu.async_copy(
            idx_hbm.at[pl.ds(wbase * ROWS_PER_EDGE + c * CHUNK_ROWS,
                             CHUNK_ROWS)],
            idx_bufs[slot], isems[slot])

    def wait_idx(slot):
        pltpu.make_async_copy(idx_hbm.at[pl.ds(0, CHUNK_ROWS)],
                              idx_bufs[slot], isems[slot]).wait()

    def fire_gather(slot):
        pltpu.async_copy(emb_sp.at[idx_bufs[slot].at[pl.ds(0, 128)]],
                         rows_bufs[slot].at[pl.ds(0, 128)], gsems[slot])
        pltpu.async_copy(emb_sp.at[idx_bufs[slot].at[pl.ds(128, 64)]],
                         rows_bufs[slot].at[pl.ds(128, 64)], gsems[slot])

    def wait_gather(slot):
        pltpu.make_async_copy(emb_hbm.at[pl.ds(0, CHUNK_ROWS)],
                              rows_bufs[slot], gsems[slot]).wait()

    def compute(c, slot):
        rows = rows_bufs[slot]
        npair = D_FEAT // 2
        zero_pair = jnp.zeros((2 * 16,), jnp.bfloat16)

        @plsc.parallel_loop(0, npair, step=UNROLL, unroll=2,
                            carry=(zero_pair,) * (1 + K_NEG))
        def accs(t, accs_in):
            out = list(accs_in)
            for u in range(UNROLL):
                dv = (t + u + iota4) & (npair - 1)
                sb = plsc.bitcast(plsc.load_gather(rows, [row_base[0], dv]),
                                  jnp.bfloat16)
                tb = plsc.bitcast(plsc.load_gather(rows, [row_base[1], dv]),
                                  jnp.bfloat16)
                out[0] = out[0] + sb * tb
                for k in range(K_NEG):
                    nb = plsc.bitcast(
                        plsc.load_gather(rows, [row_base[2 + k], dv]),
                        jnp.bfloat16)
                    out[1 + k] = out[1 + k] + sb * nb
            return tuple(out)

        def pair_sum(acc):
            even, odd = plsc.unpack(acc, format=plsc.PackFormat.INTERLEAVED,
                                    preferred_element_type=jnp.float32)
            return even + odd

        accs = [pair_sum(a) for a in accs]
        pos = accs[0]
        ssum = jnp.exp(pos * INV_T)
        for k in range(K_NEG):
            ssum = ssum + jnp.exp(accs[1 + k] * INV_T)
        os_v[pl.ds(c * CHUNK, CHUNK)] = ssum
        op_v[pl.ds(c * CHUNK, CHUNK)] = pos

    fire_idx(0, 0)
    fire_idx(1, 1)
    fire_idx(2, 2)
    wait_idx(0)
    fire_gather(0)
    wait_idx(1)
    fire_gather(1)

    n_full = (CHUNKS_PER_WORKER - 1) // NBUF

    def triple_body(i, carry):
        for par in range(NBUF):
            c = i * NBUF + par
            cur = par
            nxt = (par + 2) % NBUF
            wait_gather(cur)
            if par == 0:
                fire_idx(c + NBUF, cur)
            else:
                @pl.when(i < n_full - 1)
                def _():
                    fire_idx(c + NBUF, cur)
            if par == NBUF - 1:
                @pl.when(i < n_full - 1)
                def _():
                    wait_idx(nxt)
                    fire_gather(nxt)
            else:
                wait_idx(nxt)
                fire_gather(nxt)
            compute(c, cur)
        return carry

    lax.fori_loop(0, n_full, triple_body, 0)
    wait_gather(0)
    compute(CHUNKS_PER_WORKER - 1, 0)

    pltpu.sync_copy(os_v, out_s_hbm.at[pl.ds(wbase, EDGES_PER_WORKER)])
    pltpu.sync_copy(op_v, out_p_hbm.at[pl.ds(wbase, EDGES_PER_WORKER)])


_sc_kernel = functools.partial(
    pl.kernel,
    out_type=(jax.ShapeDtypeStruct((N_EDGES,), jnp.float32),
              jax.ShapeDtypeStruct((N_EDGES,), jnp.float32)),
    mesh=plsc.VectorSubcoreMesh(core_axis_name="c", subcore_axis_name="s",
                                num_cores=NC, num_subcores=NS),
    scratch_types=(
        [pltpu.VMEM((CHUNK_ROWS,), jnp.int32)] * NBUF
        + [pltpu.VMEM((CHUNK_ROWS, D_FEAT // 2), jnp.float32)] * NBUF
        + [pltpu.VMEM((EDGES_PER_WORKER,), jnp.float32)] * 2
        + [pltpu.VMEM_SHARED((N_NODES, D_FEAT // 2), jnp.float32)]
        + [pltpu.SemaphoreType.DMA] * (2 * NBUF)
    ),
    compiler_params=pltpu.CompilerParams(needs_layout_passes=False,
                                         use_tc_tiling_on_sc=False),
)(_sc_body)


def kernel(node_embeddings, edge_index, num_neg_samples):
    del num_neg_samples
    emb_bf16 = pl.pallas_call(
        _tc_normalize,
        out_shape=jax.ShapeDtypeStruct((N_NODES, D_FEAT), jnp.bfloat16),
    )(node_embeddings)
    emb = jax.lax.bitcast_convert_type(
        emb_bf16.reshape(N_NODES, D_FEAT // 2, 2), jnp.float32)

    src = edge_index[0].astype(jnp.int32)
    dst = edge_index[1].astype(jnp.int32)
    neg = jax.random.randint(jax.random.key(42), (N_EDGES, K_NEG), 0, N_NODES)
    idx_all = jnp.concatenate(
        [src[:, None], dst[:, None], neg.astype(jnp.int32)], axis=1
    ).reshape(-1)

    s_sum, pos = _sc_kernel(emb, idx_all)

    loss2d = pl.pallas_call(
        _tc_finalize,
        out_shape=jax.ShapeDtypeStruct((1, 1), jnp.float32),
        out_specs=pl.BlockSpec(memory_space=pltpu.SMEM),
    )(s_sum.reshape(N_EDGES // 128, 128), pos.reshape(N_EDGES // 128, 128))
    return loss2d[0, 0]


# speedup vs baseline: 1.5636x; 1.3760x over previous
import functools

import jax
import jax.numpy as jnp
from jax import lax
from jax.experimental import pallas as pl
from jax.experimental.pallas import tpu as pltpu
from jax.experimental.pallas import tpu_sc as plsc

N_NODES = 10000
D_FEAT = 128
N_EDGES = 320000
K_NEG = 10
INV_T = 2.0

NC = 2
NS = 16
NW = NC * NS
EDGES_PER_WORKER = N_EDGES // NW
CHUNK = 32
CHUNKS_PER_WORKER = -(-EDGES_PER_WORKER // CHUNK)
EDGE_PAD = CHUNKS_PER_WORKER * CHUNK - EDGES_PER_WORKER
ROWS_PER_EDGE = 2 + K_NEG
CHUNK_ROWS = CHUNK * ROWS_PER_EDGE
UNROLL = 2
NBUF = 2


def _tc_normalize(x_ref, o_ref):
    x = x_ref[...]
    n = jnp.sqrt(jnp.sum(x * x, axis=1, keepdims=True))
    o_ref[...] = (x / jnp.maximum(n, 1e-12)).astype(jnp.bfloat16)


def _tc_finalize(s_ref, p_ref, o_ref):
    tot = jnp.sum(jnp.log(s_ref[...])) - INV_T * jnp.sum(p_ref[...])
    o_ref[0, 0] = tot / N_EDGES


def _sc_body(emb_hbm, idx_hbm, out_s_hbm, out_p_hbm,
             idx_a, idx_b, rows_a, rows_b, os_v, op_v, emb_sp,
             isem_a, isem_b, gsem_a, gsem_b):
    sid = lax.axis_index("s")
    wid = sid * NC + lax.axis_index("c")
    iota4 = lax.iota(jnp.int32, 16) * 4
    wbase = wid * EDGES_PER_WORKER

    rows_per_sub = N_NODES // NS
    pltpu.sync_copy(emb_hbm.at[pl.ds(sid * rows_per_sub, rows_per_sub)],
                    emb_sp.at[pl.ds(sid * rows_per_sub, rows_per_sub)])
    plsc.subcore_barrier()
    iota = lax.iota(jnp.int32, 16)
    row_base = [iota * ROWS_PER_EDGE + j for j in range(ROWS_PER_EDGE)]
    zero = jnp.zeros((16,), jnp.float32)
    idx_bufs = (idx_a, idx_b)
    rows_bufs = (rows_a, rows_b)
    isems = (isem_a, isem_b)
    gsems = (gsem_a, gsem_b)

    def fire_idx(c, slot):
        pltpu.async_copy(
            idx_hbm.at[pl.ds(wbase * ROWS_PER_EDGE + c * CHUNK_ROWS,
                             CHUNK_ROWS)],
            idx_bufs[slot], isems[slot])

    def wait_idx(slot):
        pltpu.make_async_copy(idx_hbm.at[pl.ds(0, CHUNK_ROWS)],
                              idx_bufs[slot], isems[slot]).wait()

    def fire_gather(slot):
        for i in range(CHUNK_ROWS // 128):
            sl = pl.ds(i * 128, 128)
            pltpu.async_copy(emb_sp.at[idx_bufs[slot].at[sl]],
                             rows_bufs[slot].at[sl], gsems[slot])

    def wait_gather(slot):
        pltpu.make_async_copy(emb_hbm.at[pl.ds(0, CHUNK_ROWS)],
                              rows_bufs[slot], gsems[slot]).wait()

    def compute(c, slot):
        rows = rows_bufs[slot]
        npair = D_FEAT // 2
        zero_pair = jnp.zeros((2 * 16,), jnp.bfloat16)

        for g in range(CHUNK // 16):
            rv = [b + g * 16 * ROWS_PER_EDGE for b in row_base]

            def dbody(t, accs, rv=rv):
                out = list(accs)
                for u in range(UNROLL):
                    dv = (t * UNROLL + u + iota4) & (npair - 1)
                    sb = plsc.bitcast(plsc.load_gather(rows, [rv[0], dv]),
                                      jnp.bfloat16)
                    tb = plsc.bitcast(plsc.load_gather(rows, [rv[1], dv]),
                                      jnp.bfloat16)
                    out[0] = out[0] + sb * tb
                    for k in range(K_NEG):
                        nb = plsc.bitcast(
                            plsc.load_gather(rows, [rv[2 + k], dv]),
                            jnp.bfloat16)
                        out[1 + k] = out[1 + k] + sb * nb
                return tuple(out)

            accs = lax.fori_loop(0, npair // UNROLL, dbody,
                                 (zero_pair,) * (1 + K_NEG))

            def pair_sum(acc):
                even, odd = plsc.unpack(acc,
                                        format=plsc.PackFormat.INTERLEAVED,
                                        preferred_element_type=jnp.float32)
                return even + odd

            accs = [pair_sum(a) for a in accs]
            pos = accs[0]
            ssum = jnp.exp(pos * INV_T)
            for k in range(K_NEG):
                ssum = ssum + jnp.exp(accs[1 + k] * INV_T)
            os_v[pl.ds(c * CHUNK + g * 16, 16)] = ssum
            op_v[pl.ds(c * CHUNK + g * 16, 16)] = pos

    fire_idx(0, 0)
    fire_idx(1, 1)
    wait_idx(0)
    fire_gather(0)

    n_full = (CHUNKS_PER_WORKER - 1) // 2

    def pair_body(i, carry):
        for par in (0, 1):
            c = i * 2 + par
            cur, other = par, 1 - par
            wait_gather(cur)
            if par == 0:
                fire_idx(c + 2, cur)
            else:
                @pl.when(i < n_full - 1)
                def _():
                    fire_idx(c + 2, cur)
            wait_idx(other)
            fire_gather(other)
            compute(c, cur)
        return carry

    lax.fori_loop(0, n_full, pair_body, 0)
    wait_gather(0)
    compute(CHUNKS_PER_WORKER - 1, 0)

    n_out = EDGES_PER_WORKER + EDGE_PAD
    pltpu.sync_copy(os_v, out_s_hbm.at[pl.ds(wbase, n_out)])
    pltpu.sync_copy(op_v, out_p_hbm.at[pl.ds(wbase, n_out)])


_sc_kernel = functools.partial(
    pl.kernel,
    out_type=(jax.ShapeDtypeStruct((N_EDGES + 512,), jnp.float32),
              jax.ShapeDtypeStruct((N_EDGES + 512,), jnp.float32)),
    mesh=plsc.VectorSubcoreMesh(core_axis_name="c", subcore_axis_name="s",
                                num_cores=NC, num_subcores=NS),
    scratch_types=(
        [pltpu.VMEM((CHUNK_ROWS,), jnp.int32)] * NBUF
        + [pltpu.VMEM((CHUNK_ROWS, D_FEAT // 2), jnp.float32)] * NBUF
        + [pltpu.VMEM((EDGES_PER_WORKER + EDGE_PAD,), jnp.float32)] * 2
        + [pltpu.VMEM_SHARED((N_NODES, D_FEAT // 2), jnp.float32)]
        + [pltpu.SemaphoreType.DMA] * (2 * NBUF)
    ),
    compiler_params=pltpu.CompilerParams(needs_layout_passes=False,
                                         use_tc_tiling_on_sc=False),
)(_sc_body)


def kernel(node_embeddings, edge_index, num_neg_samples):
    del num_neg_samples
    emb_bf16 = pl.pallas_call(
        _tc_normalize,
        out_shape=jax.ShapeDtypeStruct((N_NODES, D_FEAT), jnp.bfloat16),
    )(node_embeddings)
    emb = jax.lax.bitcast_convert_type(
        emb_bf16.reshape(N_NODES, D_FEAT // 2, 2), jnp.float32)

    src = edge_index[0].astype(jnp.int32)
    dst = edge_index[1].astype(jnp.int32)
    neg = jax.random.randint(jax.random.key(42), (N_EDGES, K_NEG), 0, N_NODES)
    idx_all = jnp.concatenate(
        [src[:, None], dst[:, None], neg.astype(jnp.int32)], axis=1
    ).reshape(-1)
    idx_all = jnp.pad(idx_all, (0, 256))

    s_sum, pos = _sc_kernel(emb, idx_all)
    s_sum = s_sum[:N_EDGES]
    pos = pos[:N_EDGES]

    loss2d = pl.pallas_call(
        _tc_finalize,
        out_shape=jax.ShapeDtypeStruct((1, 1), jnp.float32),
        out_specs=pl.BlockSpec(memory_space=pltpu.SMEM),
    )(s_sum.reshape(N_EDGES // 128, 128), pos.reshape(N_EDGES // 128, 128))
    return loss2d[0, 0]


# speedup vs baseline: 1.6143x; 1.0324x over previous
import functools

import jax
import jax.numpy as jnp
from jax import lax
from jax.experimental import pallas as pl
from jax.experimental.pallas import tpu as pltpu
from jax.experimental.pallas import tpu_sc as plsc

N_NODES = 10000
D_FEAT = 128
N_EDGES = 320000
K_NEG = 10
INV_T = 2.0

NC = 2
NS = 16
NW = NC * NS
EDGES_PER_WORKER = N_EDGES // NW
CHUNK = 16
CHUNKS_PER_WORKER = -(-EDGES_PER_WORKER // CHUNK)
EDGE_PAD = CHUNKS_PER_WORKER * CHUNK - EDGES_PER_WORKER
ROWS_PER_EDGE = 2 + K_NEG
CHUNK_ROWS = CHUNK * ROWS_PER_EDGE
UNROLL = 2
NBUF = 3


def _tc_normalize(x_ref, o_ref):
    x = x_ref[...]
    n = jnp.sqrt(jnp.sum(x * x, axis=1, keepdims=True))
    o_ref[...] = (x / jnp.maximum(n, 1e-12)).astype(jnp.bfloat16)


def _tc_finalize(s_ref, p_ref, o_ref):
    tot = jnp.sum(jnp.log(s_ref[...])) - INV_T * jnp.sum(p_ref[...])
    o_ref[0, 0] = tot / N_EDGES


def _sc_body(emb_hbm, idx_hbm, out_s_hbm, out_p_hbm,
             idx_a, idx_b, idx_c, rows_a, rows_b, rows_c, os_v, op_v, emb_sp,
             isem_a, isem_b, isem_c, gsem_a, gsem_b, gsem_c):
    sid = lax.axis_index("s")
    wid = sid * NC + lax.axis_index("c")
    iota4 = lax.iota(jnp.int32, 16) * 4
    wbase = wid * EDGES_PER_WORKER

    rows_per_sub = N_NODES // NS
    pltpu.sync_copy(emb_hbm.at[pl.ds(sid * rows_per_sub, rows_per_sub)],
                    emb_sp.at[pl.ds(sid * rows_per_sub, rows_per_sub)])
    plsc.subcore_barrier()
    iota = lax.iota(jnp.int32, 16)
    row_base = [iota * ROWS_PER_EDGE + j for j in range(ROWS_PER_EDGE)]
    zero = jnp.zeros((16,), jnp.float32)
    idx_bufs = (idx_a, idx_b, idx_c)
    rows_bufs = (rows_a, rows_b, rows_c)
    isems = (isem_a, isem_b, isem_c)
    gsems = (gsem_a, gsem_b, gsem_c)

    def fire_idx(c, slot):
        pltpu.async_copy(
            idx_hbm.at[pl.ds(wbase * ROWS_PER_EDGE + c * CHUNK_ROWS,
                             CHUNK_ROWS)],
            idx_bufs[slot], isems[slot])

    def wait_idx(slot):
        pltpu.make_async_copy(idx_hbm.at[pl.ds(0, CHUNK_ROWS)],
                              idx_bufs[slot], isems[slot]).wait()

    def fire_gather(slot):
        for i in range(CHUNK_ROWS // 128):
            sl = pl.ds(i * 128, 128)
            pltpu.async_copy(emb_sp.at[idx_bufs[slot].at[sl]],
                             rows_bufs[slot].at[sl], gsems[slot])
        rem = CHUNK_ROWS % 128
        if rem:
            sl = pl.ds(CHUNK_ROWS - rem, rem)
            pltpu.async_copy(emb_sp.at[idx_bufs[slot].at[sl]],
                             rows_bufs[slot].at[sl], gsems[slot])

    def wait_gather(slot):
        pltpu.make_async_copy(emb_hbm.at[pl.ds(0, CHUNK_ROWS)],
                              rows_bufs[slot], gsems[slot]).wait()

    def compute(c, slot):
        rows = rows_bufs[slot]
        npair = D_FEAT // 2
        zero_pair = jnp.zeros((2 * 16,), jnp.bfloat16)

        for g in range(CHUNK // 16):
            rv = [b + g * 16 * ROWS_PER_EDGE for b in row_base]

            def dbody(t, accs, rv=rv):
                out = list(accs)
                for u in range(UNROLL):
                    dv = (t * UNROLL + u + iota4) & (npair - 1)
                    sb = plsc.bitcast(plsc.load_gather(rows, [rv[0], dv]),
                                      jnp.bfloat16)
                    tb = plsc.bitcast(plsc.load_gather(rows, [rv[1], dv]),
                                      jnp.bfloat16)
                    out[0] = out[0] + sb * tb
                    for k in range(K_NEG):
                        nb = plsc.bitcast(
                            plsc.load_gather(rows, [rv[2 + k], dv]),
                            jnp.bfloat16)
                        out[1 + k] = out[1 + k] + sb * nb
                return tuple(out)

            accs = lax.fori_loop(0, npair // UNROLL, dbody,
                                 (zero_pair,) * (1 + K_NEG))

            def pair_sum(acc):
                even, odd = plsc.unpack(acc,
                                        format=plsc.PackFormat.INTERLEAVED,
                                        preferred_element_type=jnp.float32)
                return even + odd

            accs = [pair_sum(a) for a in accs]
            pos = accs[0]
            ssum = jnp.exp(pos * INV_T)
            for k in range(K_NEG):
                ssum = ssum + jnp.exp(accs[1 + k] * INV_T)
            os_v[pl.ds(c * CHUNK + g * 16, 16)] = ssum
            op_v[pl.ds(c * CHUNK + g * 16, 16)] = pos

    fire_idx(0, 0)
    fire_idx(1, 1)
    fire_idx(2, 2)
    wait_idx(0)
    fire_gather(0)
    wait_idx(1)
    fire_gather(1)

    n_full = (CHUNKS_PER_WORKER - 1) // NBUF

    def triple_body(i, carry):
        for par in range(NBUF):
            c = i * NBUF + par
            cur = par
            nxt = (par + 2) % NBUF
            wait_gather(cur)
            if par == 0:
                fire_idx(c + NBUF, cur)
            else:
                @pl.when(i < n_full - 1)
                def _():
                    fire_idx(c + NBUF, cur)
            if par == NBUF - 1:
                @pl.when(i < n_full - 1)
                def _():
                    wait_idx(nxt)
                    fire_gather(nxt)
            else:
                wait_idx(nxt)
                fire_gather(nxt)
            compute(c, cur)
        return carry

    lax.fori_loop(0, n_full, triple_body, 0)
    wait_gather(0)
    compute(CHUNKS_PER_WORKER - 1, 0)

    n_out = EDGES_PER_WORKER + EDGE_PAD
    pltpu.sync_copy(os_v, out_s_hbm.at[pl.ds(wbase, n_out)])
    pltpu.sync_copy(op_v, out_p_hbm.at[pl.ds(wbase, n_out)])


_sc_kernel = functools.partial(
    pl.kernel,
    out_type=(jax.ShapeDtypeStruct((N_EDGES + 512,), jnp.float32),
              jax.ShapeDtypeStruct((N_EDGES + 512,), jnp.float32)),
    mesh=plsc.VectorSubcoreMesh(core_axis_name="c", subcore_axis_name="s",
                                num_cores=NC, num_subcores=NS),
    scratch_types=(
        [pltpu.VMEM((CHUNK_ROWS,), jnp.int32)] * NBUF
        + [pltpu.VMEM((CHUNK_ROWS, D_FEAT // 2), jnp.float32)] * NBUF
        + [pltpu.VMEM((EDGES_PER_WORKER + EDGE_PAD,), jnp.float32)] * 2
        + [pltpu.VMEM_SHARED((N_NODES, D_FEAT // 2), jnp.float32)]
        + [pltpu.SemaphoreType.DMA] * (2 * NBUF)
    ),
    compiler_params=pltpu.CompilerParams(needs_layout_passes=False,
                                         use_tc_tiling_on_sc=False),
)(_sc_body)


def kernel(node_embeddings, edge_index, num_neg_samples):
    del num_neg_samples
    emb_bf16 = pl.pallas_call(
        _tc_normalize,
        out_shape=jax.ShapeDtypeStruct((N_NODES, D_FEAT), jnp.bfloat16),
    )(node_embeddings)
    emb = jax.lax.bitcast_convert_type(
        emb_bf16.reshape(N_NODES, D_FEAT // 2, 2), jnp.float32)

    src = edge_index[0].astype(jnp.int32)
    dst = edge_index[1].astype(jnp.int32)
    neg = jax.random.randint(jax.random.key(42), (N_EDGES, K_NEG), 0, N_NODES)
    idx_all = jnp.concatenate(
        [src[:, None], dst[:, None], neg.astype(jnp.int32)], axis=1
    ).reshape(-1)
    idx_all = jnp.pad(idx_all, (0, 256))

    s_sum, pos = _sc_kernel(emb, idx_all)
    s_sum = s_sum[:N_EDGES]
    pos = pos[:N_EDGES]

    loss2d = pl.pallas_call(
        _tc_finalize,
        out_shape=jax.ShapeDtypeStruct((1, 1), jnp.float32),
        out_specs=pl.BlockSpec(memory_space=pltpu.SMEM),
    )(s_sum.reshape(N_EDGES // 128, 128), pos.reshape(N_EDGES // 128, 128))
    return loss2d[0, 0]


# speedup vs baseline: 1.6442x; 1.0186x over previous
import functools

import jax
import jax.numpy as jnp
from jax import lax
from jax.experimental import pallas as pl
from jax.experimental.pallas import tpu as pltpu
from jax.experimental.pallas import tpu_sc as plsc

N_NODES = 10000
D_FEAT = 128
N_EDGES = 320000
K_NEG = 10
INV_T = 2.0

NC = 2
NS = 16
NW = NC * NS
EDGES_PER_WORKER = N_EDGES // NW
CHUNK = 16
CHUNKS_PER_WORKER = -(-EDGES_PER_WORKER // CHUNK)
EDGE_PAD = CHUNKS_PER_WORKER * CHUNK - EDGES_PER_WORKER
ROWS_PER_EDGE = 2 + K_NEG
CHUNK_ROWS = CHUNK * ROWS_PER_EDGE
UNROLL = 2
NBUF = 3


def _tc_normalize(x_ref, o_ref):
    x = x_ref[...]
    n = jnp.sqrt(jnp.sum(x * x, axis=1, keepdims=True))
    o_ref[...] = (x / jnp.maximum(n, 1e-12)).astype(jnp.bfloat16)


def _tc_finalize(s_ref, p_ref, o_ref):
    tot = jnp.sum(jnp.log(s_ref[...])) - INV_T * jnp.sum(p_ref[...])
    o_ref[0, 0] = tot / N_EDGES


def _sc_body(emb_hbm, idx_hbm, out_s_hbm, out_p_hbm,
             idx_a, idx_b, idx_c, rows_a, rows_b, rows_c, os_v, op_v, emb_sp,
             isem_a, isem_b, isem_c, gsem_a, gsem_b, gsem_c):
    sid = lax.axis_index("s")
    wid = sid * NC + lax.axis_index("c")
    iota4 = lax.iota(jnp.int32, 16) * 4
    wbase = wid * EDGES_PER_WORKER

    rows_per_sub = N_NODES // NS
    pltpu.sync_copy(emb_hbm.at[pl.ds(sid * rows_per_sub, rows_per_sub)],
                    emb_sp.at[pl.ds(sid * rows_per_sub, rows_per_sub)])
    plsc.subcore_barrier()
    iota = lax.iota(jnp.int32, 16)
    row_base = [iota * ROWS_PER_EDGE + j for j in range(ROWS_PER_EDGE)]
    zero = jnp.zeros((16,), jnp.float32)
    idx_bufs = (idx_a, idx_b, idx_c)
    rows_bufs = (rows_a, rows_b, rows_c)
    isems = (isem_a, isem_b, isem_c)
    gsems = (gsem_a, gsem_b, gsem_c)

    def fire_idx(c, slot):
        pltpu.async_copy(
            idx_hbm.at[pl.ds(wbase * ROWS_PER_EDGE + c * CHUNK_ROWS,
                             CHUNK_ROWS)],
            idx_bufs[slot], isems[slot])

    def wait_idx(slot):
        pltpu.make_async_copy(idx_hbm.at[pl.ds(0, CHUNK_ROWS)],
                              idx_bufs[slot], isems[slot]).wait()

    def fire_gather(slot):
        for i in range(CHUNK_ROWS // 128):
            sl = pl.ds(i * 128, 128)
            pltpu.async_copy(emb_sp.at[idx_bufs[slot].at[sl]],
                             rows_bufs[slot].at[sl], gsems[slot])
        rem = CHUNK_ROWS % 128
        if rem:
            sl = pl.ds(CHUNK_ROWS - rem, rem)
            pltpu.async_copy(emb_sp.at[idx_bufs[slot].at[sl]],
                             rows_bufs[slot].at[sl], gsems[slot])

    def wait_gather(slot):
        pltpu.make_async_copy(emb_hbm.at[pl.ds(0, CHUNK_ROWS)],
                              rows_bufs[slot], gsems[slot]).wait()

    def compute(c, slot):
        rows = rows_bufs[slot]
        npair = D_FEAT // 2
        zero_pair = jnp.zeros((2 * 16,), jnp.bfloat16)

        for g in range(CHUNK // 16):
            rv = [b + g * 16 * ROWS_PER_EDGE for b in row_base]

            def dbody(t, accs, rv=rv):
                out = list(accs)
                for u in range(UNROLL):
                    dv = (t * UNROLL + u + iota4) & (npair - 1)
                    sb = plsc.bitcast(plsc.load_gather(rows, [rv[0], dv]),
                                      jnp.bfloat16)
                    tb = plsc.bitcast(plsc.load_gather(rows, [rv[1], dv]),
                                      jnp.bfloat16)
                    out[0] = out[0] + sb * tb
                    for k in range(K_NEG):
                        nb = plsc.bitcast(
                            plsc.load_gather(rows, [rv[2 + k], dv]),
                            jnp.bfloat16)
                        out[1 + k] = out[1 + k] + sb * nb
                return tuple(out)

            accs = lax.fori_loop(0, npair // UNROLL, dbody,
                                 (zero_pair,) * (1 + K_NEG))

            def pair_sum(acc):
                even, odd = plsc.unpack(acc,
                                        format=plsc.PackFormat.INTERLEAVED,
                                        preferred_element_type=jnp.float32)
                return even + odd

            accs = [pair_sum(a) for a in accs]
            pos = accs[0]
            ssum = jnp.exp(pos * INV_T)
            for k in range(K_NEG):
                ssum = ssum + jnp.exp(accs[1 + k] * INV_T)
            os_v[pl.ds(c * CHUNK + g * 16, 16)] = ssum
            op_v[pl.ds(c * CHUNK + g * 16, 16)] = pos

    fire_idx(0, 0)
    fire_idx(1, 1)
    fire_idx(2, 2)
    wait_idx(0)
    fire_gather(0)
    wait_idx(1)
    fire_gather(1)

    n_full = (CHUNKS_PER_WORKER - 1) // NBUF

    def triple_body(i, carry):
        for par in range(NBUF):
            c = i * NBUF + par
            cur = par
            nxt = (par + 2) % NBUF
            wait_gather(cur)
            if par == 0:
                fire_idx(c + NBUF, cur)
            else:
                @pl.when(i < n_full - 1)
                def _():
                    fire_idx(c + NBUF, cur)
            if par == NBUF - 1:
                @pl.when(i < n_full - 1)
                def _():
                    wait_idx(nxt)
                    fire_gather(nxt)
            else:
                wait_idx(nxt)
                fire_gather(nxt)
            compute(c, cur)
        return carry

    lax.fori_loop(0, n_full, triple_body, 0)
    wait_gather(0)
    compute(CHUNKS_PER_WORKER - 1, 0)

    n_out = EDGES_PER_WORKER + EDGE_PAD
    pltpu.sync_copy(os_v, out_s_hbm.at[pl.ds(wbase, n_out)])
    pltpu.sync_copy(op_v, out_p_hbm.at[pl.ds(wbase, n_out)])


_sc_kernel = functools.partial(
    pl.kernel,
    out_type=(jax.ShapeDtypeStruct((N_EDGES + (512 if EDGE_PAD else 0),),
                                   jnp.float32),) * 2,
    mesh=plsc.VectorSubcoreMesh(core_axis_name="c", subcore_axis_name="s",
                                num_cores=NC, num_subcores=NS),
    scratch_types=(
        [pltpu.VMEM((CHUNK_ROWS,), jnp.int32)] * NBUF
        + [pltpu.VMEM((CHUNK_ROWS, D_FEAT // 2), jnp.float32)] * NBUF
        + [pltpu.VMEM((EDGES_PER_WORKER + EDGE_PAD,), jnp.float32)] * 2
        + [pltpu.VMEM_SHARED((N_NODES, D_FEAT // 2), jnp.float32)]
        + [pltpu.SemaphoreType.DMA] * (2 * NBUF)
    ),
    compiler_params=pltpu.CompilerParams(needs_layout_passes=False,
                                         use_tc_tiling_on_sc=False),
)(_sc_body)


def kernel(node_embeddings, edge_index, num_neg_samples):
    del num_neg_samples
    emb_bf16 = pl.pallas_call(
        _tc_normalize,
        out_shape=jax.ShapeDtypeStruct((N_NODES, D_FEAT), jnp.bfloat16),
    )(node_embeddings)
    emb = jax.lax.bitcast_convert_type(
        emb_bf16.reshape(N_NODES, D_FEAT // 2, 2), jnp.float32)

    src = edge_index[0].astype(jnp.int32)
    dst = edge_index[1].astype(jnp.int32)
    neg = jax.random.randint(jax.random.key(42), (N_EDGES, K_NEG), 0, N_NODES)
    idx_all = jnp.concatenate(
        [src[:, None], dst[:, None], neg.astype(jnp.int32)], axis=1
    ).reshape(-1)
    if EDGE_PAD:
        idx_all = jnp.pad(idx_all, (0, 256))

    s_sum, pos = _sc_kernel(emb, idx_all)
    if EDGE_PAD:
        s_sum = s_sum[:N_EDGES]
        pos = pos[:N_EDGES]

    loss2d = pl.pallas_call(
        _tc_finalize,
        out_shape=jax.ShapeDtypeStruct((1, 1), jnp.float32),
        out_specs=pl.BlockSpec(memory_space=pltpu.SMEM),
    )(s_sum.reshape(N_EDGES // 128, 128), pos.reshape(N_EDGES // 128, 128))
    return loss2d[0, 0]


# speedup vs baseline: 1.6572x; 1.0079x over previous
import functools

import jax
import jax.numpy as jnp
from jax import lax
from jax.experimental import pallas as pl
from jax.experimental.pallas import tpu as pltpu
from jax.experimental.pallas import tpu_sc as plsc

N_NODES = 10000
D_FEAT = 128
N_EDGES = 320000
K_NEG = 10
INV_T = 2.0

NC = 2
NS = 16
NW = NC * NS
EDGES_PER_WORKER = N_EDGES // NW
CHUNK = 16
CHUNKS_PER_WORKER = -(-EDGES_PER_WORKER // CHUNK)
EDGE_PAD = CHUNKS_PER_WORKER * CHUNK - EDGES_PER_WORKER
ROWS_PER_EDGE = 2 + K_NEG
CHUNK_ROWS = CHUNK * ROWS_PER_EDGE
UNROLL = 1
NBUF = 3


def _tc_normalize(x_ref, o_ref):
    x = x_ref[...]
    n = jnp.sqrt(jnp.sum(x * x, axis=1, keepdims=True))
    o_ref[...] = (x / jnp.maximum(n, 1e-12)).astype(jnp.bfloat16)


def _tc_finalize(s_ref, p_ref, o_ref):
    tot = jnp.sum(jnp.log(s_ref[...])) - INV_T * jnp.sum(p_ref[...])
    o_ref[0, 0] = tot / N_EDGES


def _sc_body(emb_hbm, idx_hbm, out_s_hbm, out_p_hbm,
             idx_a, idx_b, idx_c, rows_a, rows_b, rows_c, os_v, op_v, emb_sp,
             isem_a, isem_b, isem_c, gsem_a, gsem_b, gsem_c):
    sid = lax.axis_index("s")
    wid = sid * NC + lax.axis_index("c")
    iota4 = lax.iota(jnp.int32, 16) * 4
    wbase = wid * EDGES_PER_WORKER

    rows_per_sub = N_NODES // NS
    pltpu.sync_copy(emb_hbm.at[pl.ds(sid * rows_per_sub, rows_per_sub)],
                    emb_sp.at[pl.ds(sid * rows_per_sub, rows_per_sub)])
    plsc.subcore_barrier()
    iota = lax.iota(jnp.int32, 16)
    row_base = [iota * ROWS_PER_EDGE + j for j in range(ROWS_PER_EDGE)]
    zero = jnp.zeros((16,), jnp.float32)
    idx_bufs = (idx_a, idx_b, idx_c)
    rows_bufs = (rows_a, rows_b, rows_c)
    isems = (isem_a, isem_b, isem_c)
    gsems = (gsem_a, gsem_b, gsem_c)

    def fire_idx(c, slot):
        pltpu.async_copy(
            idx_hbm.at[pl.ds(wbase * ROWS_PER_EDGE + c * CHUNK_ROWS,
                             CHUNK_ROWS)],
            idx_bufs[slot], isems[slot])

    def wait_idx(slot):
        pltpu.make_async_copy(idx_hbm.at[pl.ds(0, CHUNK_ROWS)],
                              idx_bufs[slot], isems[slot]).wait()

    def fire_gather(slot):
        for i in range(CHUNK_ROWS // 128):
            sl = pl.ds(i * 128, 128)
            pltpu.async_copy(emb_sp.at[idx_bufs[slot].at[sl]],
                             rows_bufs[slot].at[sl], gsems[slot])
        rem = CHUNK_ROWS % 128
        if rem:
            sl = pl.ds(CHUNK_ROWS - rem, rem)
            pltpu.async_copy(emb_sp.at[idx_bufs[slot].at[sl]],
                             rows_bufs[slot].at[sl], gsems[slot])

    def wait_gather(slot):
        pltpu.make_async_copy(emb_hbm.at[pl.ds(0, CHUNK_ROWS)],
                              rows_bufs[slot], gsems[slot]).wait()

    def compute(c, slot):
        rows = rows_bufs[slot]
        npair = D_FEAT // 2
        zero_pair = jnp.zeros((2 * 16,), jnp.bfloat16)

        for g in range(CHUNK // 16):
            rv = [b + g * 16 * ROWS_PER_EDGE for b in row_base]

            def dbody(t, accs, rv=rv):
                out = list(accs)
                for u in range(UNROLL):
                    dv = (t * UNROLL + u + iota4) & (npair - 1)
                    sb = plsc.bitcast(plsc.load_gather(rows, [rv[0], dv]),
                                      jnp.bfloat16)
                    tb = plsc.bitcast(plsc.load_gather(rows, [rv[1], dv]),
                                      jnp.bfloat16)
                    out[0] = out[0] + sb * tb
                    for k in range(K_NEG):
                        nb = plsc.bitcast(
                            plsc.load_gather(rows, [rv[2 + k], dv]),
                            jnp.bfloat16)
                        out[1 + k] = out[1 + k] + sb * nb
                return tuple(out)

            accs = lax.fori_loop(0, npair // UNROLL, dbody,
                                 (zero_pair,) * (1 + K_NEG))

            def pair_sum(acc):
                even, odd = plsc.unpack(acc,
                                        format=plsc.PackFormat.INTERLEAVED,
                                        preferred_element_type=jnp.float32)
                return even + odd

            accs = [pair_sum(a) for a in accs]
            pos = accs[0]
            ssum = jnp.exp(pos * INV_T)
            for k in range(K_NEG):
                ssum = ssum + jnp.exp(accs[1 + k] * INV_T)
            os_v[pl.ds(c * CHUNK + g * 16, 16)] = ssum
            op_v[pl.ds(c * CHUNK + g * 16, 16)] = pos

    fire_idx(0, 0)
    fire_idx(1, 1)
    fire_idx(2, 2)
    wait_idx(0)
    fire_gather(0)
    wait_idx(1)
    fire_gather(1)

    n_full = (CHUNKS_PER_WORKER - 1) // NBUF

    def triple_body(i, carry):
        for par in range(NBUF):
            c = i * NBUF + par
            cur = par
            nxt = (par + 2) % NBUF
            wait_gather(cur)
            if par == 0:
                fire_idx(c + NBUF, cur)
            else:
                @pl.when(i < n_full - 1)
                def _():
                    fire_idx(c + NBUF, cur)
            if par == NBUF - 1:
                @pl.when(i < n_full - 1)
                def _():
                    wait_idx(nxt)
                    fire_gather(nxt)
            else:
                wait_idx(nxt)
                fire_gather(nxt)
            compute(c, cur)
        return carry

    lax.fori_loop(0, n_full, triple_body, 0)
    wait_gather(0)
    compute(CHUNKS_PER_WORKER - 1, 0)

    n_out = EDGES_PER_WORKER + EDGE_PAD
    pltpu.sync_copy(os_v, out_s_hbm.at[pl.ds(wbase, n_out)])
    pltpu.sync_copy(op_v, out_p_hbm.at[pl.ds(wbase, n_out)])


_sc_kernel = functools.partial(
    pl.kernel,
    out_type=(jax.ShapeDtypeStruct((N_EDGES + (512 if EDGE_PAD else 0),),
                                   jnp.float32),) * 2,
    mesh=plsc.VectorSubcoreMesh(core_axis_name="c", subcore_axis_name="s",
                                num_cores=NC, num_subcores=NS),
    scratch_types=(
        [pltpu.VMEM((CHUNK_ROWS,), jnp.int32)] * NBUF
        + [pltpu.VMEM((CHUNK_ROWS, D_FEAT // 2), jnp.float32)] * NBUF
        + [pltpu.VMEM((EDGES_PER_WORKER + EDGE_PAD,), jnp.float32)] * 2
        + [pltpu.VMEM_SHARED((N_NODES, D_FEAT // 2), jnp.float32)]
        + [pltpu.SemaphoreType.DMA] * (2 * NBUF)
    ),
    compiler_params=pltpu.CompilerParams(needs_layout_passes=False,
                                         use_tc_tiling_on_sc=False),
)(_sc_body)


def kernel(node_embeddings, edge_index, num_neg_samples):
    del num_neg_samples
    emb_bf16 = pl.pallas_call(
        _tc_normalize,
        out_shape=jax.ShapeDtypeStruct((N_NODES, D_FEAT), jnp.bfloat16),
    )(node_embeddings)
    emb = jax.lax.bitcast_convert_type(
        emb_bf16.reshape(N_NODES, D_FEAT // 2, 2), jnp.float32)

    src = edge_index[0].astype(jnp.int32)
    dst = edge_index[1].astype(jnp.int32)
    neg = jax.random.randint(jax.random.key(42), (N_EDGES, K_NEG), 0, N_NODES)
    idx_all = jnp.concatenate(
        [src[:, None], dst[:, None], neg.astype(jnp.int32)], axis=1
    ).reshape(-1)
    if EDGE_PAD:
        idx_all = jnp.pad(idx_all, (0, 256))

    s_sum, pos = _sc_kernel(emb, idx_all)
    if EDGE_PAD:
        s_sum = s_sum[:N_EDGES]
        pos = pos[:N_EDGES]

    loss2d = pl.pallas_call(
        _tc_finalize,
        out_shape=jax.ShapeDtypeStruct((1, 1), jnp.float32),
        out_specs=pl.BlockSpec(memory_space=pltpu.SMEM),
    )(s_sum.reshape(N_EDGES // 128, 128), pos.reshape(N_EDGES // 128, 128))
    return loss2d[0, 0]
